# baseline (speedup 1.0000x reference)
import functools

import jax
import jax.numpy as jnp
from jax import lax
from jax.experimental import pallas as pl
from jax.experimental.pallas import tpu as pltpu
from jax.experimental.pallas import tpu_sc as plsc

HOP = 5
CH = 48
RING = 8
GRP = RING * CH
N_TILES = 16
N_CORES = 2
BCH = 3136
FB = 2 * GRP


def _bin_body(cfg, src_hbm, dst_hbm, val_hbm,
              srcb, dstb, valb, cnts,
              inS0, inD0, inV0, inS1, inD1, inV1,
              obufS, obufD, obufV, cntv, counter,
              sti0, sti1, sof):
    half, cpt, _cap = cfg
    c = lax.axis_index("c")
    s = lax.axis_index("s")
    base = c * half
    inS = (inS0, inS1)
    inD = (inD0, inD1)
    inV = (inV0, inV1)
    sti = (sti0, sti1)
    ebase = s * cpt * BCH

    counter[0] = 0
    plsc.subcore_barrier()

    def fire_d(i, b):
        pltpu.async_copy(dst_hbm.at[pl.ds(ebase + i * BCH, BCH)], inD[b],
                         sti[b])

    def wait_d(b):
        pltpu.make_async_copy(dst_hbm.at[pl.ds(ebase, BCH)], inD[b],
                              sti[b]).wait()

    fire_d(0, 0)
    fire_d(1, 1)

    def count_pair(i2, cv):
        for b in range(2):
            i = i2 * 2 + b
            wait_d(b)

            def cgrp(q, cv):
                dv = inD[b][pl.ds(q * 16, 16)]
                m = (dv >= base) & (dv < base + half)
                return cv + jnp.where(m, 1, 0)
            cv = lax.fori_loop(0, BCH // 16, cgrp, cv)

            @pl.when(i + 2 < cpt)
            def _():
                fire_d(i + 2, b)
        return cv
    cv = lax.fori_loop(0, cpt // 2, count_pair,
                       jnp.zeros((16,), jnp.int32))
    total = jnp.sum(cv)
    target = -(-total // FB) * FB
    off = plsc.fetch_and_add(counter.at[0], target, subcore_id=0)

    def fire_sdv(i, b):
        e0 = ebase + i * BCH
        pltpu.async_copy(src_hbm.at[pl.ds(e0, BCH)], inS[b], sti[b])
        pltpu.async_copy(dst_hbm.at[pl.ds(e0, BCH)], inD[b], sti[b])
        pltpu.async_copy(val_hbm.at[pl.ds(e0, BCH)], inV[b], sti[b])

    def wait_sdv(b):
        for hbm, buf in ((src_hbm, inS[b]), (dst_hbm, inD[b]),
                         (val_hbm, inV[b])):
            pltpu.make_async_copy(hbm.at[pl.ds(ebase, BCH)], buf,
                                  sti[b]).wait()

    trips = ((obufS, srcb), (obufD, dstb), (obufV, valb))

    def emit(sv, dv, vv, m, carry):
        o, wr, nf = carry
        plsc.store_compressed(obufS.at[pl.ds(o, 16)], sv, mask=m)
        plsc.store_compressed(obufD.at[pl.ds(o, 16)], dv, mask=m)
        plsc.store_compressed(obufV.at[pl.ds(o, 16)], vv, mask=m)
        pc = plsc.all_reduce_population_count(m)[0]
        o = o + pc
        wr = wr + pc
        flush = wr >= (nf + 1) * FB

        @pl.when(flush)
        def _():
            @pl.when(nf >= 1)
            def _():
                for buf, hbm in trips:
                    pltpu.make_async_copy(buf.at[pl.ds(0, FB)],
                                          hbm.at[c, pl.ds(0, FB)], sof).wait()
            fo = pl.multiple_of((nf % 2) * FB, FB)
            ho = pl.multiple_of(off + nf * FB, FB)
            for buf, hbm in trips:
                pltpu.async_copy(buf.at[pl.ds(fo, FB)],
                                 hbm.at[c, pl.ds(ho, FB)], sof)
        nf = jnp.where(flush, nf + 1, nf)
        wrap = o >= 2 * FB

        @pl.when(wrap)
        def _():
            for buf, _ in trips:
                buf[pl.ds(0, 16)] = buf[pl.ds(2 * FB, 16)]
        o = jnp.where(wrap, o - 2 * FB, o)
        return (o, wr, nf)

    fire_sdv(0, 0)
    fire_sdv(1, 1)

    def comp_pair(i2, carry):
        for b in range(2):
            i = i2 * 2 + b
            wait_sdv(b)

            def cgrp(q, carry):
                sl = pl.ds(q * 16, 16)
                sv = inS[b][sl]
                dv = inD[b][sl]
                vv = inV[b][sl]
                m = (dv >= base) & (dv < base + half)
                return emit(sv, dv, vv, m, carry)
            carry = lax.fori_loop(0, BCH // 16, cgrp, carry)

            @pl.when(i + 2 < cpt)
            def _():
                fire_sdv(i + 2, b)
        return carry
    carry = lax.fori_loop(0, cpt // 2, comp_pair,
                          (jnp.int32(0), jnp.int32(0), jnp.int32(0)))
    o, wr, nf = carry

    dummy_s = jnp.zeros((16,), jnp.int32)
    dummy_d = jnp.zeros((16,), jnp.int32) + base
    dummy_v = jnp.zeros((16,), jnp.float32)
    p1 = (16 - wr % 16) % 16
    lanes = lax.broadcasted_iota(jnp.int32, (16,), 0)
    o, wr, nf = emit(dummy_s, dummy_d, dummy_v, lanes < p1, (o, wr, nf))

    def padg(k, carry):
        m = jnp.zeros((16,), jnp.int32) + jnp.where(carry[1] < target, 1, 0)
        return emit(dummy_s, dummy_d, dummy_v, m > 0, carry)
    o, wr, nf = lax.fori_loop(0, 2 * FB // 16, padg, (o, wr, nf))

    @pl.when(nf >= 1)
    def _():
        for buf, hbm in trips:
            pltpu.make_async_copy(buf.at[pl.ds(0, FB)],
                                  hbm.at[c, pl.ds(0, FB)], sof).wait()
    plsc.subcore_barrier()

    @pl.when(s == 0)
    def _():
        cntv[pl.ds(0, 16)] = jnp.zeros((16,), jnp.int32) + counter[0]
        pltpu.sync_copy(cntv, cnts.at[c])


@functools.partial(jax.jit, static_argnums=(3,))
def _bin_sc(src1d, dst1d, val1d, cfgb):
    half, cpt, cap = cfgb
    mesh = plsc.VectorSubcoreMesh(core_axis_name="c", subcore_axis_name="s",
                                  num_cores=N_CORES, num_subcores=N_TILES)
    kern = pl.kernel(
        functools.partial(_bin_body, cfgb),
        out_type=[
            jax.ShapeDtypeStruct((N_CORES, cap), jnp.int32),
            jax.ShapeDtypeStruct((N_CORES, cap), jnp.int32),
            jax.ShapeDtypeStruct((N_CORES, cap), jnp.float32),
            jax.ShapeDtypeStruct((N_CORES, 16), jnp.int32),
        ],
        mesh=mesh,
        scratch_types=[
            pltpu.VMEM((BCH,), jnp.int32),
            pltpu.VMEM((BCH,), jnp.int32),
            pltpu.VMEM((BCH,), jnp.float32),
            pltpu.VMEM((BCH,), jnp.int32),
            pltpu.VMEM((BCH,), jnp.int32),
            pltpu.VMEM((BCH,), jnp.float32),
            pltpu.VMEM((2 * FB + 16,), jnp.int32),
            pltpu.VMEM((2 * FB + 16,), jnp.int32),
            pltpu.VMEM((2 * FB + 16,), jnp.float32),
            pltpu.VMEM((16,), jnp.int32),
            pltpu.SMEM((1,), jnp.int32),
            pltpu.SemaphoreType.DMA,
            pltpu.SemaphoreType.DMA,
            pltpu.SemaphoreType.DMA,
        ],
        compiler_params=pltpu.CompilerParams(use_tc_tiling_on_sc=False,
                                             needs_layout_passes=False),
    )
    return kern(src1d, dst1d, val1d)


def _spmm_body(cfg, h_hbm, srcb, dstb, valb, cnts, zeros_hbm, out_hbm,
               acc, cbuf,
               srcS0, dstS0, valS0, srcS1, dstS1, valS1, dl, rows,
               *sems):
    half, rpt, feat = cfg
    c = lax.axis_index("c")
    s = lax.axis_index("s")
    base = c * half
    srcS = (srcS0, srcS1)
    dstS = (dstS0, dstS1)
    valS = (valS0, valS1)
    sgs = sems[:RING]
    sss = sems[RING:2 * RING]
    sts = sems[2 * RING:]

    pltpu.sync_copy(cnts.at[c], cbuf)
    total = cbuf[pl.ds(0, 16)][0]
    g_total = total // GRP
    gpt = (g_total + N_TILES - 1) // N_TILES
    gcnt = jnp.clip(g_total - s * gpt, 0, gpt)
    g0 = jnp.where(gcnt > 0, s * gpt, 0)
    n_sub = gcnt * RING

    def stage_group(g, st, sync):
        e0 = pl.multiple_of((g0 + g) * GRP, GRP)
        for hbm, buf in ((srcb, srcS[st]), (dstb, dstS[st]),
                         (valb, valS[st])):
            if sync:
                pltpu.sync_copy(hbm.at[c, pl.ds(e0, GRP)], buf)
            else:
                pltpu.async_copy(hbm.at[c, pl.ds(e0, GRP)], buf, sts[st])

    def wait_stage(st):
        for hbm, buf in ((srcb, srcS[st]), (dstb, dstS[st]),
                         (valb, valS[st])):
            pltpu.make_async_copy(hbm.at[c, pl.ds(0, GRP)], buf,
                                  sts[st]).wait()

    def slot(ref, k):
        return ref.at[pl.ds(k * CH, CH)]

    def fire_gather(st, j, k):
        pltpu.async_copy(h_hbm.at[srcS[st].at[pl.ds(j * CH, CH)]],
                         slot(rows, k), sgs[k])

    def wait_gather(k):
        pltpu.make_async_copy(h_hbm.at[srcS[0].at[pl.ds(0, CH)]],
                              slot(rows, k), sgs[k]).wait()

    def fire_scatter(k):
        pltpu.async_copy(slot(rows, k), acc.at[dl.at[k]], sss[k], add=True)

    def wait_scatter(k):
        pltpu.make_async_copy(slot(rows, k), acc.at[dl.at[k]], sss[k]).wait()

    pltpu.sync_copy(zeros_hbm, acc.at[pl.ds(s * rpt, rpt)])

    @pl.when(gcnt > 0)
    def _():
        stage_group(0, 0, sync=True)
        for _j in range(RING - 1):
            fire_gather(0, _j, _j)
    plsc.subcore_barrier()

    def do_group(g, gg):
        @pl.when(g + 1 < gcnt)
        def _():
            stage_group(g + 1, 1 - gg, sync=False)

        for j in range(RING):
            t = g * RING + j

            for q in range(CH // 16):
                d = dstS[gg][pl.ds(j * CH + q * 16, 16)] - base
                ok = (d >= 0) & (d < half)
                dl[j, pl.ds(q * 16, 16)] = jnp.where(ok, d, half)

            wait_gather(j)

            @plsc.parallel_loop(0, CH // 16, 1, unroll=2)
            def _(e16):
                vv = valS[gg][pl.ds(j * CH + e16 * 16, 16)]
                for u in range(16):
                    e = j * CH + e16 * 16 + u
                    v = vv[u]
                    for fb in range(feat // 16):
                        rows[e, pl.ds(fb * 16, 16)] = (
                            rows[e, pl.ds(fb * 16, 16)] * v)

            fire_scatter(j)

            k2 = (j + RING - 1) % RING
            @pl.when(t >= 1)
            def _():
                wait_scatter(k2)

            @pl.when(t + RING - 1 < n_sub)
            def _():
                if j == 0:
                    fire_gather(gg, RING - 1, k2)
                else:
                    if j == 1:
                        wait_stage(1 - gg)
                    fire_gather(1 - gg, j - 1, k2)

    def loop_body(g2, _):
        do_group(g2 * 2, 0)
        do_group(g2 * 2 + 1, 1)
        return 0
    lax.fori_loop(0, gcnt // 2, loop_body, 0)

    @pl.when(gcnt % 2 == 1)
    def _():
        do_group(gcnt - 1, 0)

    @pl.when(gcnt > 0)
    def _():
        wait_scatter(RING - 1)
    plsc.subcore_barrier()

    start = jnp.minimum(s * rpt, half - rpt)
    pltpu.sync_copy(acc.at[pl.ds(start, rpt)],
                    out_hbm.at[pl.ds(base + start, rpt)])


@functools.partial(jax.jit, static_argnums=(6,))
def _spmm_sc(h, srcb, dstb, valb, cnts, zeros, cfg):
    n, feat = h.shape
    half, rpt, _ = cfg
    mesh = plsc.VectorSubcoreMesh(core_axis_name="c", subcore_axis_name="s",
                                  num_cores=N_CORES, num_subcores=N_TILES)
    kern = pl.kernel(
        functools.partial(_spmm_body, cfg),
        out_type=jax.ShapeDtypeStruct((n, feat), jnp.float32),
        mesh=mesh,
        scratch_types=[
            pltpu.VMEM_SHARED((rpt * N_TILES, feat), jnp.float32),
            pltpu.VMEM((16,), jnp.int32),
            pltpu.VMEM((GRP,), jnp.int32),
            pltpu.VMEM((GRP,), jnp.int32),
            pltpu.VMEM((GRP,), jnp.float32),
            pltpu.VMEM((GRP,), jnp.int32),
            pltpu.VMEM((GRP,), jnp.int32),
            pltpu.VMEM((GRP,), jnp.float32),
            pltpu.VMEM((RING, CH), jnp.int32),
            pltpu.VMEM((RING * CH, feat), jnp.float32),
        ] + [pltpu.SemaphoreType.DMA] * (2 * RING + 2),
        compiler_params=pltpu.CompilerParams(use_tc_tiling_on_sc=False,
                                             needs_layout_passes=False),
    )
    return kern(h, srcb, dstb, valb, cnts, zeros)


def _proj_body(x_ref, w_ref, b_ref, o_ref):
    o_ref[...] = jax.nn.relu(
        jnp.dot(x_ref[...], w_ref[...], preferred_element_type=jnp.float32)
        + b_ref[...])


def _proj(x, w, b, blk):
    n, d_in = x.shape
    h = w.shape[1]
    return pl.pallas_call(
        _proj_body,
        grid=(n // blk,),
        in_specs=[
            pl.BlockSpec((blk, d_in), lambda i: (i, 0)),
            pl.BlockSpec((d_in, h), lambda i: (0, 0)),
            pl.BlockSpec((h,), lambda i: (0,)),
        ],
        out_specs=pl.BlockSpec((blk, h), lambda i: (i, 0)),
        out_shape=jax.ShapeDtypeStruct((n, h), jnp.float32),
    )(x, w, b)


def _scores_body(m0_ref, m1_ref, x_ref, d0_ref, d1_ref, w_ref, b_ref, q_ref,
                 z0_ref, z1_ref, wsum_ref, acc):
    z0 = m0_ref[...] + d0_ref[...] * x_ref[...]
    z1 = m1_ref[...] + d1_ref[...] * x_ref[...]
    z0_ref[...] = z0
    z1_ref[...] = z1
    w = w_ref[...]
    b = b_ref[...]
    q = q_ref[...]
    t0 = jnp.tanh(jnp.dot(z0, w, preferred_element_type=jnp.float32) + b)
    t1 = jnp.tanh(jnp.dot(z1, w, preferred_element_type=jnp.float32) + b)
    s0 = jnp.sum(t0 * q[:, 0])
    s1 = jnp.sum(t1 * q[:, 0])

    @pl.when(pl.program_id(0) == 0)
    def _():
        acc[0] = 0.0
        acc[1] = 0.0

    acc[0] += s0
    acc[1] += s1

    @pl.when(pl.program_id(0) == pl.num_programs(0) - 1)
    def _():
        wsum_ref[0] = acc[0]
        wsum_ref[1] = acc[1]


def _scores(m0, m1, x, d0, d1, saW, sab, saq, blk):
    n, h = x.shape
    hid = saW.shape[1]
    return pl.pallas_call(
        _scores_body,
        grid=(n // blk,),
        in_specs=[
            pl.BlockSpec((blk, h), lambda i: (i, 0)),
            pl.BlockSpec((blk, h), lambda i: (i, 0)),
            pl.BlockSpec((blk, h), lambda i: (i, 0)),
            pl.BlockSpec((blk, 1), lambda i: (i, 0)),
            pl.BlockSpec((blk, 1), lambda i: (i, 0)),
            pl.BlockSpec((h, hid), lambda i: (0, 0)),
            pl.BlockSpec((hid,), lambda i: (0,)),
            pl.BlockSpec((hid, 1), lambda i: (0, 0)),
        ],
        out_specs=[
            pl.BlockSpec((blk, h), lambda i: (i, 0)),
            pl.BlockSpec((blk, h), lambda i: (i, 0)),
            pl.BlockSpec(memory_space=pltpu.SMEM),
        ],
        out_shape=[
            jax.ShapeDtypeStruct((n, h), jnp.float32),
            jax.ShapeDtypeStruct((n, h), jnp.float32),
            jax.ShapeDtypeStruct((2,), jnp.float32),
        ],
        scratch_shapes=[pltpu.SMEM((2,), jnp.float32)],
    )(m0, m1, x, d0, d1, saW, sab, saq)


def _combine_body(n_nodes, z0_ref, z1_ref, wsum_ref, o_ref):
    w0 = wsum_ref[0] / n_nodes
    w1 = wsum_ref[1] / n_nodes
    m = jnp.maximum(w0, w1)
    e0 = jnp.exp(w0 - m)
    e1 = jnp.exp(w1 - m)
    b0 = e0 / (e0 + e1)
    b1 = e1 / (e0 + e1)
    o_ref[...] = b0 * z0_ref[...] + b1 * z1_ref[...]


def _combine(z0, z1, wsum, blk):
    n, h = z0.shape
    return pl.pallas_call(
        functools.partial(_combine_body, float(n)),
        grid=(n // blk,),
        in_specs=[
            pl.BlockSpec((blk, h), lambda i: (i, 0)),
            pl.BlockSpec((blk, h), lambda i: (i, 0)),
            pl.BlockSpec(memory_space=pltpu.SMEM),
        ],
        out_specs=pl.BlockSpec((blk, h), lambda i: (i, 0)),
        out_shape=jax.ShapeDtypeStruct((n, h), jnp.float32),
    )(z0, z1, wsum)


def _final_body(h_ref, w_ref, b_ref, o_ref):
    o_ref[...] = (jnp.dot(h_ref[...], w_ref[...],
                          preferred_element_type=jnp.float32) + b_ref[...])


def _final(hp, w2, b2, blk):
    n, h = hp.shape
    out = w2.shape[1]
    return pl.pallas_call(
        _final_body,
        grid=(n // blk,),
        in_specs=[
            pl.BlockSpec((blk, h), lambda i: (i, 0)),
            pl.BlockSpec((h, out), lambda i: (0, 0)),
            pl.BlockSpec((out,), lambda i: (0,)),
        ],
        out_specs=pl.BlockSpec((blk, out), lambda i: (i, 0)),
        out_shape=jax.ShapeDtypeStruct((n, out), jnp.float32),
    )(hp, w2, b2)


def _prep_edges(src, dst, val, e_pad):
    e = src.shape[0]
    pad = e_pad - e
    src = jnp.concatenate([src.astype(jnp.int32), jnp.zeros((pad,), jnp.int32)])
    dst = jnp.concatenate([dst.astype(jnp.int32), jnp.zeros((pad,), jnp.int32)])
    val = jnp.concatenate([val, jnp.zeros((pad,), jnp.float32)])
    return src, dst, val


def kernel(x_paper, x_author, src_pp, dst_pp, val_pp, diag_pp,
           src_pa, dst_pa, val_pa, diag_pa,
           src_aa, dst_aa, val_aa, diag_aa,
           src_ap, dst_ap, val_ap, diag_ap,
           W1_paper, b1_paper, W1_author, b1_author,
           saW_paper, sab_paper, saq_paper,
           saW_author, sab_author, saq_author,
           W2, b2):
    n, _ = x_paper.shape
    feat = W1_paper.shape[1]
    e = src_pp.shape[0]
    half = n // 2
    rpt = -(-(-(-(half + 1) // N_TILES)) // 8) * 8
    cpt = max(2, 2 * (-(-e // (N_TILES * BCH * 2))))
    e_pad = N_TILES * cpt * BCH
    cap = N_TILES * (-(-(cpt * BCH) // FB) * FB)
    cfgb = (half, cpt, cap)
    cfg = (half, rpt, feat)
    blk = 2000

    zeros = jnp.zeros((rpt, feat), jnp.float32)
    edges = {}
    for rel, (sr, ds_, vl) in {
        "pp": (src_pp, dst_pp, val_pp),
        "pa": (src_pa, dst_pa, val_pa),
        "aa": (src_aa, dst_aa, val_aa),
        "ap": (src_ap, dst_ap, val_ap),
    }.items():
        edges[rel] = _bin_sc(*_prep_edges(sr, ds_, vl, e_pad), cfgb)

    x_p = _proj(x_paper, W1_paper, b1_paper, blk)
    x_a = _proj(x_author, W1_author, b1_author, blk)
    h_p, h_a = x_p, x_a
    for _ in range(HOP):
        m0 = _spmm_sc(h_p, *edges["pp"], zeros, cfg)
        m1 = _spmm_sc(h_a, *edges["pa"], zeros, cfg)
        m0a = _spmm_sc(h_a, *edges["aa"], zeros, cfg)
        z0, z1, wsum = _scores(m0, m1, x_p, diag_pp, diag_pa,
                               saW_paper, sab_paper, saq_paper, blk)
        h_p = _combine(z0, z1, wsum, blk)
        m0 = m0a
        m1 = _spmm_sc(h_p, *edges["ap"], zeros, cfg)
        z0, z1, wsum = _scores(m0, m1, x_a, diag_aa, diag_ap,
                               saW_author, sab_author, saq_author, blk)
        h_a = _combine(z0, z1, wsum, blk)
    return _final(h_p, W2, b2, blk)


# speedup vs baseline: 1.0544x; 1.0544x over previous
import functools

import jax
import jax.numpy as jnp
from jax import lax
from jax.experimental import pallas as pl
from jax.experimental.pallas import tpu as pltpu
from jax.experimental.pallas import tpu_sc as plsc

HOP = 5
CH = 64
RING = 6
GRP = RING * CH
N_TILES = 16
N_CORES = 2
BCH = 3136
FB = 2 * GRP


def _bin_body(cfg, src_hbm, dst_hbm, val_hbm,
              srcb, dstb, valb, cnts,
              inS0, inD0, inV0, inS1, inD1, inV1,
              obufS, obufD, obufV, cntv, counter,
              sti0, sti1, sof):
    half, cpt, _cap = cfg
    c = lax.axis_index("c")
    s = lax.axis_index("s")
    base = c * half
    inS = (inS0, inS1)
    inD = (inD0, inD1)
    inV = (inV0, inV1)
    sti = (sti0, sti1)
    ebase = s * cpt * BCH

    counter[0] = 0
    plsc.subcore_barrier()

    def fire_d(i, b):
        pltpu.async_copy(dst_hbm.at[pl.ds(ebase + i * BCH, BCH)], inD[b],
                         sti[b])

    def wait_d(b):
        pltpu.make_async_copy(dst_hbm.at[pl.ds(ebase, BCH)], inD[b],
                              sti[b]).wait()

    fire_d(0, 0)
    fire_d(1, 1)

    def count_pair(i2, cv):
        for b in range(2):
            i = i2 * 2 + b
            wait_d(b)

            def cgrp(q, cv):
                dv = inD[b][pl.ds(q * 16, 16)]
                m = (dv >= base) & (dv < base + half)
                return cv + jnp.where(m, 1, 0)
            cv = lax.fori_loop(0, BCH // 16, cgrp, cv)

            @pl.when(i + 2 < cpt)
            def _():
                fire_d(i + 2, b)
        return cv
    cv = lax.fori_loop(0, cpt // 2, count_pair,
                       jnp.zeros((16,), jnp.int32))
    total = jnp.sum(cv)
    target = -(-total // FB) * FB
    off = plsc.fetch_and_add(counter.at[0], target, subcore_id=0)

    def fire_sdv(i, b):
        e0 = ebase + i * BCH
        pltpu.async_copy(src_hbm.at[pl.ds(e0, BCH)], inS[b], sti[b])
        pltpu.async_copy(dst_hbm.at[pl.ds(e0, BCH)], inD[b], sti[b])
        pltpu.async_copy(val_hbm.at[pl.ds(e0, BCH)], inV[b], sti[b])

    def wait_sdv(b):
        for hbm, buf in ((src_hbm, inS[b]), (dst_hbm, inD[b]),
                         (val_hbm, inV[b])):
            pltpu.make_async_copy(hbm.at[pl.ds(ebase, BCH)], buf,
                                  sti[b]).wait()

    trips = ((obufS, srcb), (obufD, dstb), (obufV, valb))

    def emit(sv, dv, vv, m, carry):
        o, wr, nf = carry
        plsc.store_compressed(obufS.at[pl.ds(o, 16)], sv, mask=m)
        plsc.store_compressed(obufD.at[pl.ds(o, 16)], dv, mask=m)
        plsc.store_compressed(obufV.at[pl.ds(o, 16)], vv, mask=m)
        pc = plsc.all_reduce_population_count(m)[0]
        o = o + pc
        wr = wr + pc
        flush = wr >= (nf + 1) * FB

        @pl.when(flush)
        def _():
            @pl.when(nf >= 1)
            def _():
                for buf, hbm in trips:
                    pltpu.make_async_copy(buf.at[pl.ds(0, FB)],
                                          hbm.at[c, pl.ds(0, FB)], sof).wait()
            fo = pl.multiple_of((nf % 2) * FB, FB)
            ho = pl.multiple_of(off + nf * FB, FB)
            for buf, hbm in trips:
                pltpu.async_copy(buf.at[pl.ds(fo, FB)],
                                 hbm.at[c, pl.ds(ho, FB)], sof)
        nf = jnp.where(flush, nf + 1, nf)
        wrap = o >= 2 * FB

        @pl.when(wrap)
        def _():
            for buf, _ in trips:
                buf[pl.ds(0, 16)] = buf[pl.ds(2 * FB, 16)]
        o = jnp.where(wrap, o - 2 * FB, o)
        return (o, wr, nf)

    fire_sdv(0, 0)
    fire_sdv(1, 1)

    def comp_pair(i2, carry):
        for b in range(2):
            i = i2 * 2 + b
            wait_sdv(b)

            def cgrp(q, carry):
                sl = pl.ds(q * 16, 16)
                sv = inS[b][sl]
                dv = inD[b][sl]
                vv = inV[b][sl]
                m = (dv >= base) & (dv < base + half)
                return emit(sv, dv, vv, m, carry)
            carry = lax.fori_loop(0, BCH // 16, cgrp, carry)

            @pl.when(i + 2 < cpt)
            def _():
                fire_sdv(i + 2, b)
        return carry
    carry = lax.fori_loop(0, cpt // 2, comp_pair,
                          (jnp.int32(0), jnp.int32(0), jnp.int32(0)))
    o, wr, nf = carry

    dummy_s = jnp.zeros((16,), jnp.int32)
    dummy_d = jnp.zeros((16,), jnp.int32) + base
    dummy_v = jnp.zeros((16,), jnp.float32)
    p1 = (16 - wr % 16) % 16
    lanes = lax.broadcasted_iota(jnp.int32, (16,), 0)
    o, wr, nf = emit(dummy_s, dummy_d, dummy_v, lanes < p1, (o, wr, nf))

    def padg(k, carry):
        m = jnp.zeros((16,), jnp.int32) + jnp.where(carry[1] < target, 1, 0)
        return emit(dummy_s, dummy_d, dummy_v, m > 0, carry)
    o, wr, nf = lax.fori_loop(0, 2 * FB // 16, padg, (o, wr, nf))

    @pl.when(nf >= 1)
    def _():
        for buf, hbm in trips:
            pltpu.make_async_copy(buf.at[pl.ds(0, FB)],
                                  hbm.at[c, pl.ds(0, FB)], sof).wait()
    plsc.subcore_barrier()

    @pl.when(s == 0)
    def _():
        cntv[pl.ds(0, 16)] = jnp.zeros((16,), jnp.int32) + counter[0]
        pltpu.sync_copy(cntv, cnts.at[c])


@functools.partial(jax.jit, static_argnums=(3,))
def _bin_sc(src1d, dst1d, val1d, cfgb):
    half, cpt, cap = cfgb
    mesh = plsc.VectorSubcoreMesh(core_axis_name="c", subcore_axis_name="s",
                                  num_cores=N_CORES, num_subcores=N_TILES)
    kern = pl.kernel(
        functools.partial(_bin_body, cfgb),
        out_type=[
            jax.ShapeDtypeStruct((N_CORES, cap), jnp.int32),
            jax.ShapeDtypeStruct((N_CORES, cap), jnp.int32),
            jax.ShapeDtypeStruct((N_CORES, cap), jnp.float32),
            jax.ShapeDtypeStruct((N_CORES, 16), jnp.int32),
        ],
        mesh=mesh,
        scratch_types=[
            pltpu.VMEM((BCH,), jnp.int32),
            pltpu.VMEM((BCH,), jnp.int32),
            pltpu.VMEM((BCH,), jnp.float32),
            pltpu.VMEM((BCH,), jnp.int32),
            pltpu.VMEM((BCH,), jnp.int32),
            pltpu.VMEM((BCH,), jnp.float32),
            pltpu.VMEM((2 * FB + 16,), jnp.int32),
            pltpu.VMEM((2 * FB + 16,), jnp.int32),
            pltpu.VMEM((2 * FB + 16,), jnp.float32),
            pltpu.VMEM((16,), jnp.int32),
            pltpu.SMEM((1,), jnp.int32),
            pltpu.SemaphoreType.DMA,
            pltpu.SemaphoreType.DMA,
            pltpu.SemaphoreType.DMA,
        ],
        compiler_params=pltpu.CompilerParams(use_tc_tiling_on_sc=False,
                                             needs_layout_passes=False),
    )
    return kern(src1d, dst1d, val1d)


def _spmm_body(cfg, h_hbm, srcb, dstb, valb, cnts, zeros_hbm, out_hbm,
               acc, cbuf,
               srcS0, dstS0, valS0, srcS1, dstS1, valS1, dl, rows,
               *sems):
    half, rpt, feat = cfg
    c = lax.axis_index("c")
    s = lax.axis_index("s")
    base = c * half
    srcS = (srcS0, srcS1)
    dstS = (dstS0, dstS1)
    valS = (valS0, valS1)
    sgs = sems[:RING]
    sss = sems[RING:2 * RING]
    sts = sems[2 * RING:]

    pltpu.sync_copy(cnts.at[c], cbuf)
    total = cbuf[pl.ds(0, 16)][0]
    g_total = total // GRP
    gpt = (g_total + N_TILES - 1) // N_TILES
    gcnt = jnp.clip(g_total - s * gpt, 0, gpt)
    g0 = jnp.where(gcnt > 0, s * gpt, 0)
    n_sub = gcnt * RING

    def stage_group(g, st, sync):
        e0 = pl.multiple_of((g0 + g) * GRP, GRP)
        for hbm, buf in ((srcb, srcS[st]), (dstb, dstS[st]),
                         (valb, valS[st])):
            if sync:
                pltpu.sync_copy(hbm.at[c, pl.ds(e0, GRP)], buf)
            else:
                pltpu.async_copy(hbm.at[c, pl.ds(e0, GRP)], buf, sts[st])

    def wait_stage(st):
        for hbm, buf in ((srcb, srcS[st]), (dstb, dstS[st]),
                         (valb, valS[st])):
            pltpu.make_async_copy(hbm.at[c, pl.ds(0, GRP)], buf,
                                  sts[st]).wait()

    def slot(ref, k):
        return ref.at[pl.ds(k * CH, CH)]

    def fire_gather(st, j, k):
        pltpu.async_copy(h_hbm.at[srcS[st].at[pl.ds(j * CH, CH)]],
                         slot(rows, k), sgs[k])

    def wait_gather(k):
        pltpu.make_async_copy(h_hbm.at[srcS[0].at[pl.ds(0, CH)]],
                              slot(rows, k), sgs[k]).wait()

    def fire_scatter(k):
        pltpu.async_copy(slot(rows, k), acc.at[dl.at[k]], sss[k], add=True)

    def wait_scatter(k):
        pltpu.make_async_copy(slot(rows, k), acc.at[dl.at[k]], sss[k]).wait()

    pltpu.sync_copy(zeros_hbm, acc.at[pl.ds(s * rpt, rpt)])

    @pl.when(gcnt > 0)
    def _():
        stage_group(0, 0, sync=True)
        for _j in range(RING - 1):
            fire_gather(0, _j, _j)
    plsc.subcore_barrier()

    def do_group(g, gg):
        @pl.when(g + 1 < gcnt)
        def _():
            stage_group(g + 1, 1 - gg, sync=False)

        for j in range(RING):
            t = g * RING + j

            for q in range(CH // 16):
                d = dstS[gg][pl.ds(j * CH + q * 16, 16)] - base
                ok = (d >= 0) & (d < half)
                dl[j, pl.ds(q * 16, 16)] = jnp.where(ok, d, half)

            wait_gather(j)

            @plsc.parallel_loop(0, CH // 16, 1, unroll=2)
            def _(e16):
                vv = valS[gg][pl.ds(j * CH + e16 * 16, 16)]
                for u in range(16):
                    e = j * CH + e16 * 16 + u
                    v = vv[u]
                    for fb in range(feat // 16):
                        rows[e, pl.ds(fb * 16, 16)] = (
                            rows[e, pl.ds(fb * 16, 16)] * v)

            fire_scatter(j)

            k2 = (j + RING - 1) % RING
            @pl.when(t >= 1)
            def _():
                wait_scatter(k2)

            @pl.when(t + RING - 1 < n_sub)
            def _():
                if j == 0:
                    fire_gather(gg, RING - 1, k2)
                else:
                    if j == 1:
                        wait_stage(1 - gg)
                    fire_gather(1 - gg, j - 1, k2)

    def loop_body(g2, _):
        do_group(g2 * 2, 0)
        do_group(g2 * 2 + 1, 1)
        return 0
    lax.fori_loop(0, gcnt // 2, loop_body, 0)

    @pl.when(gcnt % 2 == 1)
    def _():
        do_group(gcnt - 1, 0)

    @pl.when(gcnt > 0)
    def _():
        wait_scatter(RING - 1)
    plsc.subcore_barrier()

    start = jnp.minimum(s * rpt, half - rpt)
    pltpu.sync_copy(acc.at[pl.ds(start, rpt)],
                    out_hbm.at[pl.ds(base + start, rpt)])


@functools.partial(jax.jit, static_argnums=(6,))
def _spmm_sc(h, srcb, dstb, valb, cnts, zeros, cfg):
    n, feat = h.shape
    half, rpt, _ = cfg
    mesh = plsc.VectorSubcoreMesh(core_axis_name="c", subcore_axis_name="s",
                                  num_cores=N_CORES, num_subcores=N_TILES)
    kern = pl.kernel(
        functools.partial(_spmm_body, cfg),
        out_type=jax.ShapeDtypeStruct((n, feat), jnp.float32),
        mesh=mesh,
        scratch_types=[
            pltpu.VMEM_SHARED((rpt * N_TILES, feat), jnp.float32),
            pltpu.VMEM((16,), jnp.int32),
            pltpu.VMEM((GRP,), jnp.int32),
            pltpu.VMEM((GRP,), jnp.int32),
            pltpu.VMEM((GRP,), jnp.float32),
            pltpu.VMEM((GRP,), jnp.int32),
            pltpu.VMEM((GRP,), jnp.int32),
            pltpu.VMEM((GRP,), jnp.float32),
            pltpu.VMEM((RING, CH), jnp.int32),
            pltpu.VMEM((RING * CH, feat), jnp.float32),
        ] + [pltpu.SemaphoreType.DMA] * (2 * RING + 2),
        compiler_params=pltpu.CompilerParams(use_tc_tiling_on_sc=False,
                                             needs_layout_passes=False),
    )
    return kern(h, srcb, dstb, valb, cnts, zeros)


def _proj_body(x_ref, w_ref, b_ref, o_ref):
    o_ref[...] = jax.nn.relu(
        jnp.dot(x_ref[...], w_ref[...], preferred_element_type=jnp.float32)
        + b_ref[...])


def _proj(x, w, b, blk):
    n, d_in = x.shape
    h = w.shape[1]
    return pl.pallas_call(
        _proj_body,
        grid=(n // blk,),
        in_specs=[
            pl.BlockSpec((blk, d_in), lambda i: (i, 0)),
            pl.BlockSpec((d_in, h), lambda i: (0, 0)),
            pl.BlockSpec((h,), lambda i: (0,)),
        ],
        out_specs=pl.BlockSpec((blk, h), lambda i: (i, 0)),
        out_shape=jax.ShapeDtypeStruct((n, h), jnp.float32),
    )(x, w, b)


def _scores_body(m0_ref, m1_ref, x_ref, d0_ref, d1_ref, w_ref, b_ref, q_ref,
                 z0_ref, z1_ref, wsum_ref, acc):
    z0 = m0_ref[...] + d0_ref[...] * x_ref[...]
    z1 = m1_ref[...] + d1_ref[...] * x_ref[...]
    z0_ref[...] = z0
    z1_ref[...] = z1
    w = w_ref[...]
    b = b_ref[...]
    q = q_ref[...]
    t0 = jnp.tanh(jnp.dot(z0, w, preferred_element_type=jnp.float32) + b)
    t1 = jnp.tanh(jnp.dot(z1, w, preferred_element_type=jnp.float32) + b)
    s0 = jnp.sum(t0 * q[:, 0])
    s1 = jnp.sum(t1 * q[:, 0])

    @pl.when(pl.program_id(0) == 0)
    def _():
        acc[0] = 0.0
        acc[1] = 0.0

    acc[0] += s0
    acc[1] += s1

    @pl.when(pl.program_id(0) == pl.num_programs(0) - 1)
    def _():
        wsum_ref[0] = acc[0]
        wsum_ref[1] = acc[1]


def _scores(m0, m1, x, d0, d1, saW, sab, saq, blk):
    n, h = x.shape
    hid = saW.shape[1]
    return pl.pallas_call(
        _scores_body,
        grid=(n // blk,),
        in_specs=[
            pl.BlockSpec((blk, h), lambda i: (i, 0)),
            pl.BlockSpec((blk, h), lambda i: (i, 0)),
            pl.BlockSpec((blk, h), lambda i: (i, 0)),
            pl.BlockSpec((blk, 1), lambda i: (i, 0)),
            pl.BlockSpec((blk, 1), lambda i: (i, 0)),
            pl.BlockSpec((h, hid), lambda i: (0, 0)),
            pl.BlockSpec((hid,), lambda i: (0,)),
            pl.BlockSpec((hid, 1), lambda i: (0, 0)),
        ],
        out_specs=[
            pl.BlockSpec((blk, h), lambda i: (i, 0)),
            pl.BlockSpec((blk, h), lambda i: (i, 0)),
            pl.BlockSpec(memory_space=pltpu.SMEM),
        ],
        out_shape=[
            jax.ShapeDtypeStruct((n, h), jnp.float32),
            jax.ShapeDtypeStruct((n, h), jnp.float32),
            jax.ShapeDtypeStruct((2,), jnp.float32),
        ],
        scratch_shapes=[pltpu.SMEM((2,), jnp.float32)],
    )(m0, m1, x, d0, d1, saW, sab, saq)


def _combine_body(n_nodes, z0_ref, z1_ref, wsum_ref, o_ref):
    w0 = wsum_ref[0] / n_nodes
    w1 = wsum_ref[1] / n_nodes
    m = jnp.maximum(w0, w1)
    e0 = jnp.exp(w0 - m)
    e1 = jnp.exp(w1 - m)
    b0 = e0 / (e0 + e1)
    b1 = e1 / (e0 + e1)
    o_ref[...] = b0 * z0_ref[...] + b1 * z1_ref[...]


def _combine(z0, z1, wsum, blk):
    n, h = z0.shape
    return pl.pallas_call(
        functools.partial(_combine_body, float(n)),
        grid=(n // blk,),
        in_specs=[
            pl.BlockSpec((blk, h), lambda i: (i, 0)),
            pl.BlockSpec((blk, h), lambda i: (i, 0)),
            pl.BlockSpec(memory_space=pltpu.SMEM),
        ],
        out_specs=pl.BlockSpec((blk, h), lambda i: (i, 0)),
        out_shape=jax.ShapeDtypeStruct((n, h), jnp.float32),
    )(z0, z1, wsum)


def _final_body(h_ref, w_ref, b_ref, o_ref):
    o_ref[...] = (jnp.dot(h_ref[...], w_ref[...],
                          preferred_element_type=jnp.float32) + b_ref[...])


def _final(hp, w2, b2, blk):
    n, h = hp.shape
    out = w2.shape[1]
    return pl.pallas_call(
        _final_body,
        grid=(n // blk,),
        in_specs=[
            pl.BlockSpec((blk, h), lambda i: (i, 0)),
            pl.BlockSpec((h, out), lambda i: (0, 0)),
            pl.BlockSpec((out,), lambda i: (0,)),
        ],
        out_specs=pl.BlockSpec((blk, out), lambda i: (i, 0)),
        out_shape=jax.ShapeDtypeStruct((n, out), jnp.float32),
    )(hp, w2, b2)


def _prep_edges(src, dst, val, e_pad):
    e = src.shape[0]
    pad = e_pad - e
    src = jnp.concatenate([src.astype(jnp.int32), jnp.zeros((pad,), jnp.int32)])
    dst = jnp.concatenate([dst.astype(jnp.int32), jnp.zeros((pad,), jnp.int32)])
    val = jnp.concatenate([val, jnp.zeros((pad,), jnp.float32)])
    return src, dst, val


def kernel(x_paper, x_author, src_pp, dst_pp, val_pp, diag_pp,
           src_pa, dst_pa, val_pa, diag_pa,
           src_aa, dst_aa, val_aa, diag_aa,
           src_ap, dst_ap, val_ap, diag_ap,
           W1_paper, b1_paper, W1_author, b1_author,
           saW_paper, sab_paper, saq_paper,
           saW_author, sab_author, saq_author,
           W2, b2):
    n, _ = x_paper.shape
    feat = W1_paper.shape[1]
    e = src_pp.shape[0]
    half = n // 2
    rpt = -(-(-(-(half + 1) // N_TILES)) // 8) * 8
    cpt = max(2, 2 * (-(-e // (N_TILES * BCH * 2))))
    e_pad = N_TILES * cpt * BCH
    cap = N_TILES * (-(-(cpt * BCH) // FB) * FB)
    cfgb = (half, cpt, cap)
    cfg = (half, rpt, feat)
    blk = 2000

    zeros = jnp.zeros((rpt, feat), jnp.float32)
    edges = {}
    for rel, (sr, ds_, vl) in {
        "pp": (src_pp, dst_pp, val_pp),
        "pa": (src_pa, dst_pa, val_pa),
        "aa": (src_aa, dst_aa, val_aa),
        "ap": (src_ap, dst_ap, val_ap),
    }.items():
        edges[rel] = _bin_sc(*_prep_edges(sr, ds_, vl, e_pad), cfgb)

    x_p = _proj(x_paper, W1_paper, b1_paper, blk)
    x_a = _proj(x_author, W1_author, b1_author, blk)
    h_p, h_a = x_p, x_a
    for _ in range(HOP):
        m0 = _spmm_sc(h_p, *edges["pp"], zeros, cfg)
        m1 = _spmm_sc(h_a, *edges["pa"], zeros, cfg)
        m0a = _spmm_sc(h_a, *edges["aa"], zeros, cfg)
        z0, z1, wsum = _scores(m0, m1, x_p, diag_pp, diag_pa,
                               saW_paper, sab_paper, saq_paper, blk)
        h_p = _combine(z0, z1, wsum, blk)
        m0 = m0a
        m1 = _spmm_sc(h_p, *edges["ap"], zeros, cfg)
        z0, z1, wsum = _scores(m0, m1, x_a, diag_aa, diag_ap,
                               saW_author, sab_author, saq_author, blk)
        h_a = _combine(z0, z1, wsum, blk)
    return _final(h_p, W2, b2, blk)
